# baseline (speedup 1.0000x reference)
import functools

import jax
import jax.numpy as jnp
from jax import lax
from jax.experimental import pallas as pl
from jax.experimental.pallas import tpu as pltpu
from jax.experimental.pallas import tpu_sc as plsc

NU = 50000
NI = 50000
E = 800000
DIN = 128
H = 64
DOUT = 32
W = 32

EP = 819200
ER = EP // 128
NP = 50048
DUMP = 50000
NTILE = 16
EPT = EP // 2 // NTILE
CH = 320
NCH = EPT // CH
CHC = 640
NCHC = EPT // CHC
TS = NP // NTILE

BLK = 2000


def _sc_segsum(npass):
  mesh = plsc.VectorSubcoreMesh(core_axis_name="c", subcore_axis_name="s")
  out1 = jax.ShapeDtypeStruct((2 * NP, W), jnp.float32)
  buf = lambda: [pltpu.VMEM((CH,), jnp.int32),
                 pltpu.VMEM((CH,), jnp.int32),
                 pltpu.VMEM((CH, W), jnp.float32),
                 pltpu.SemaphoreType.DMA,
                 pltpu.SemaphoreType.DMA]

  @functools.partial(
      pl.kernel,
      out_type=tuple(out1 for _ in range(npass)),
      mesh=mesh,
      compiler_params=pltpu.CompilerParams(use_tc_tiling_on_sc=False),
      scratch_types=buf() + buf() + [
          pltpu.VMEM_SHARED((NP, W), jnp.float32),
      ],
  )
  def k(*refs):
    ins = refs[:3 * npass + 1]
    zeros_hbm = ins[-1]
    outs = refs[3 * npass + 1:4 * npass + 1]
    (srcv0, dstv0, rows0, gsem0, ssem0,
     srcv1, dstv1, rows1, gsem1, ssem1, acc) = refs[4 * npass + 1:]
    c = lax.axis_index("c")
    s = lax.axis_index("s")
    bufs = ((srcv0, dstv0, rows0, gsem0, ssem0),
            (srcv1, dstv1, rows1, gsem1, ssem1))

    def drain_sc(rows, ssem):
      pltpu.make_async_copy(rows, acc.at[pl.ds(0, CH)], ssem).wait()

    for p in range(npass):
      y, src2, dst2, out = ins[3 * p], ins[3 * p + 1], ins[3 * p + 2], outs[p]
      pltpu.sync_copy(zeros_hbm.at[pl.ds(0, TS)], acc.at[pl.ds(s * TS, TS)])
      plsc.subcore_barrier()
      base = c * (EP // 2) + s * EPT

      def pair(i2, carry):
        gcps = []
        for b, (srcv, dstv, rows, gsem, ssem) in enumerate(bufs):
          ci = 2 * i2 + b

          @pl.when(i2 > 0)
          def _():
            drain_sc(rows, ssem)

          pltpu.sync_copy(src2.at[pl.ds(base + ci * CH, CH)], srcv)
          pltpu.sync_copy(dst2.at[pl.ds(base + ci * CH, CH)], dstv)
          gcps.append(pltpu.async_copy(y.at[srcv], rows, gsem))
        for b, (srcv, dstv, rows, gsem, ssem) in enumerate(bufs):
          gcps[b].wait()
          pltpu.async_copy(rows, acc.at[dstv], ssem, add=True)
        return carry

      lax.fori_loop(0, NCH // 2, pair, 0)
      for (srcv, dstv, rows, gsem, ssem) in bufs:
        drain_sc(rows, ssem)
      plsc.subcore_barrier()
      pltpu.sync_copy(acc.at[pl.ds(s * TS, TS)],
                      out.at[pl.ds(c * NP + s * TS, TS)])
    return

  return k


def _sc_counts():
  mesh = plsc.VectorSubcoreMesh(core_axis_name="c", subcore_axis_name="s")
  out1 = jax.ShapeDtypeStruct((2 * NP, 16), jnp.float32)

  @functools.partial(
      pl.kernel,
      out_type=(out1, out1, out1),
      mesh=mesh,
      compiler_params=pltpu.CompilerParams(use_tc_tiling_on_sc=False),
      scratch_types=[
          pltpu.VMEM((CHC,), jnp.int32),
          pltpu.VMEM((CHC,), jnp.int32),
          pltpu.VMEM((CHC, 16), jnp.float32),
          pltpu.VMEM_SHARED((NP, 16), jnp.float32),
          pltpu.SemaphoreType.DMA,
          pltpu.SemaphoreType.DMA,
      ],
  )
  def k(dsta, dstb, dstc, ones_hbm, zeros_hbm,
        outa, outb, outc, dstv0, dstv1, ones, acc, sem0, sem1):
    c = lax.axis_index("c")
    s = lax.axis_index("s")
    pltpu.sync_copy(ones_hbm, ones)
    bufs = ((dstv0, sem0), (dstv1, sem1))

    def drain(sem):
      pltpu.make_async_copy(ones, acc.at[pl.ds(0, CHC)], sem).wait()

    for dst2, out in ((dsta, outa), (dstb, outb), (dstc, outc)):
      pltpu.sync_copy(zeros_hbm.at[pl.ds(0, TS)], acc.at[pl.ds(s * TS, TS)])
      plsc.subcore_barrier()
      base = c * (EP // 2) + s * EPT

      def pair(i2, carry):
        for b, (dstv, sem) in enumerate(bufs):
          @pl.when(i2 > 0)
          def _():
            drain(sem)

          pltpu.sync_copy(dst2.at[pl.ds(base + (2 * i2 + b) * CHC, CHC)], dstv)
          pltpu.async_copy(ones, acc.at[dstv], sem, add=True)
        return carry

      lax.fori_loop(0, NCHC // 2, pair, 0)
      for dstv, sem in bufs:
        drain(sem)
      plsc.subcore_barrier()
      pltpu.sync_copy(acc.at[pl.ds(s * TS, TS)],
                      out.at[pl.ds(c * NP + s * TS, TS)])
    return

  return k


_HI = lax.Precision.HIGHEST


def _full(shape):
  return pl.BlockSpec(shape, lambda i: (0,) * len(shape))


def _dot(a, b):
  return jnp.dot(a, b, preferred_element_type=jnp.float32, precision=_HI)


def _tc_pre(n, ny):
  grid = n // BLK
  in_specs = [pl.BlockSpec((BLK, DIN), lambda i: (i, 0)),
              _full((DIN, H)), _full((1, H))]
  in_specs += [_full((H, H))] * ny
  in_specs += [_full((H, H)), _full((1, H))]
  out_shape = tuple([jax.ShapeDtypeStruct((2, n, W), jnp.float32)] * ny
                    + [jax.ShapeDtypeStruct((n, H), jnp.float32)])
  out_specs = tuple([pl.BlockSpec((2, BLK, W), lambda i: (0, i, 0))] * ny
                    + [pl.BlockSpec((BLK, H), lambda i: (i, 0))])

  def body(*refs):
    x, pwt, pb = refs[0], refs[1], refs[2]
    wls = refs[3:3 + ny]
    wrt, blc = refs[3 + ny], refs[4 + ny]
    youts = refs[5 + ny:5 + 2 * ny]
    duo = refs[5 + 2 * ny]
    h = _dot(x[...], pwt[...]) + pb[...]
    for wl, yo in zip(wls, youts):
      yv = _dot(h, wl[...])
      yo[0] = yv[:, :W]
      yo[1] = yv[:, W:]
    duo[...] = _dot(h, wrt[...]) + blc[...]

  return pl.pallas_call(body, grid=(grid,), in_specs=in_specs,
                        out_specs=out_specs, out_shape=out_shape)


def _cnt_inv(cnt_ref):
  return 1.0 / jnp.maximum(cnt_ref[0, :, 0:1] + cnt_ref[1, :, 0:1], 1.0)


def _tc_mid(n, nrel, ny):
  grid = n // BLK
  in_specs = []
  for _ in range(nrel):
    in_specs += [pl.BlockSpec((2, BLK, W), lambda i: (0, i, 0)),
                 pl.BlockSpec((2, BLK, W), lambda i: (0, i, 0)),
                 pl.BlockSpec((2, BLK, 16), lambda i: (0, i, 0))]
  in_specs += [pl.BlockSpec((BLK, H), lambda i: (i, 0))]
  in_specs += [_full((H, DOUT))] * ny
  in_specs += [_full((H, DOUT)), _full((1, DOUT))]
  out_shape = tuple([jax.ShapeDtypeStruct((n, DOUT), jnp.float32)] * ny
                    + [jax.ShapeDtypeStruct((n, DOUT), jnp.float32)])
  out_specs = tuple([pl.BlockSpec((BLK, DOUT), lambda i: (i, 0))] * (ny + 1))
  scale = 1.0 / nrel

  def body(*refs):
    pre = None
    for r in range(nrel):
      pa, pb, cn = refs[3 * r], refs[3 * r + 1], refs[3 * r + 2]
      agg = jnp.concatenate([pa[0] + pa[1], pb[0] + pb[1]],
                            axis=1) * _cnt_inv(cn)
      pre = agg if pre is None else pre + agg
    d = refs[3 * nrel]
    wls = refs[3 * nrel + 1:3 * nrel + 1 + ny]
    wrt, blc = refs[3 * nrel + 1 + ny], refs[3 * nrel + 2 + ny]
    youts = refs[3 * nrel + 3 + ny:3 * nrel + 3 + 2 * ny]
    duo = refs[3 * nrel + 3 + 2 * ny]
    h1 = jnp.maximum((pre + d[...]) * scale, 0.0)
    for wl, yo in zip(wls, youts):
      yo[...] = _dot(h1, wl[...])
    duo[...] = _dot(h1, wrt[...]) + blc[...]

  return pl.pallas_call(body, grid=(grid,), in_specs=in_specs,
                        out_specs=out_specs, out_shape=out_shape)


def _tc_post(n, nrel):
  grid = n // BLK
  in_specs = []
  for _ in range(nrel):
    in_specs += [pl.BlockSpec((2, BLK, W), lambda i: (0, i, 0)),
                 pl.BlockSpec((2, BLK, 16), lambda i: (0, i, 0))]
  in_specs += [pl.BlockSpec((BLK, DOUT), lambda i: (i, 0))]
  out_shape = jax.ShapeDtypeStruct((n, DOUT), jnp.float32)
  out_specs = pl.BlockSpec((BLK, DOUT), lambda i: (i, 0))
  scale = 1.0 / nrel

  def body(*refs):
    pre = None
    for r in range(nrel):
      p, cn = refs[2 * r], refs[2 * r + 1]
      agg = (p[0] + p[1]) * _cnt_inv(cn)
      pre = agg if pre is None else pre + agg
    d = refs[2 * nrel]
    refs[2 * nrel + 1][...] = jnp.maximum((pre + d[...]) * scale, 0.0)

  return pl.pallas_call(body, grid=(grid,), in_specs=in_specs,
                        out_specs=out_specs, out_shape=out_shape)


def _prep_edges(ei, n_src):
  src = ei[0].astype(jnp.int32)
  dst = ei[1].astype(jnp.int32)
  pad = EP - E
  src = jnp.concatenate([src, jnp.zeros((pad,), jnp.int32)])
  dst = jnp.concatenate([dst, jnp.full((pad,), DUMP, jnp.int32)])
  return src, src + n_src, dst


def kernel(x_user, x_item, edge_index_social, edge_index_interacts,
           edge_index_rev_interacts, up_W, up_b, ip_W, ip_b,
           c1s_Wl, c1s_bl, c1s_Wr, c1i_Wl, c1i_bl, c1i_Wr,
           c1r_Wl, c1r_bl, c1r_Wr,
           c2s_Wl, c2s_bl, c2s_Wr, c2i_Wl, c2i_bl, c2i_Wr,
           c2r_Wl, c2r_bl, c2r_Wr):
  sA_s, sB_s, d_s = _prep_edges(edge_index_social, NU)
  sA_i, sB_i, d_i = _prep_edges(edge_index_interacts, NU)
  sA_r, sB_r, d_r = _prep_edges(edge_index_rev_interacts, NI)

  zeros32 = jnp.zeros((TS, W), jnp.float32)
  zeros16 = jnp.zeros((TS, 16), jnp.float32)
  ones128 = jnp.ones((CHC, 16), jnp.float32)

  ys1, yi1, du1 = _tc_pre(NU, 2)(
      x_user, up_W.T, up_b.reshape(1, H),
      c1s_Wl.T, c1i_Wl.T,
      (c1s_Wr + c1r_Wr).T, (c1s_bl + c1r_bl).reshape(1, H))
  yr1, di1 = _tc_pre(NI, 1)(
      x_item, ip_W.T, ip_b.reshape(1, H),
      c1r_Wl.T,
      c1i_Wr.T, c1i_bl.reshape(1, H))

  cnt_s, cnt_i, cnt_r = _sc_counts()(d_s, d_i, d_r, ones128, zeros16)
  ys1f = ys1.reshape(2 * NU, W)
  yi1f = yi1.reshape(2 * NU, W)
  yr1f = yr1.reshape(2 * NI, W)
  sa_s, sb_s, sa_i, sb_i, sa_r, sb_r = _sc_segsum(6)(
      ys1f, sA_s, d_s, ys1f, sB_s, d_s,
      yi1f, sA_i, d_i, yi1f, sB_i, d_i,
      yr1f, sA_r, d_r, yr1f, sB_r, d_r,
      zeros32)

  r3 = lambda a: a.reshape(2, NP, W)
  rc = lambda a: a.reshape(2, NP, 16)
  ys2, yi2, du2 = _tc_mid(NU, 2, 2)(
      r3(sa_s), r3(sb_s), rc(cnt_s),
      r3(sa_r), r3(sb_r), rc(cnt_r),
      du1,
      c2s_Wl.T, c2i_Wl.T,
      (c2s_Wr + c2r_Wr).T, (c2s_bl + c2r_bl).reshape(1, DOUT))
  yr2, di2 = _tc_mid(NI, 1, 1)(
      r3(sa_i), r3(sb_i), rc(cnt_i),
      di1,
      c2r_Wl.T,
      c2i_Wr.T, c2i_bl.reshape(1, DOUT))

  s2_s, s2_i, s2_r = _sc_segsum(3)(
      ys2, sA_s, d_s,
      yi2, sA_i, d_i,
      yr2, sA_r, d_r,
      zeros32)

  h2u = _tc_post(NU, 2)(
      r3(s2_s), rc(cnt_s),
      r3(s2_r), rc(cnt_r),
      du2)
  h2i = _tc_post(NI, 1)(
      r3(s2_i), rc(cnt_i),
      di2)
  return (h2u, h2i)


# speedup vs baseline: 1.1458x; 1.1458x over previous
import functools

import jax
import jax.numpy as jnp
from jax import lax
from jax.experimental import pallas as pl
from jax.experimental.pallas import tpu as pltpu
from jax.experimental.pallas import tpu_sc as plsc

NU = 50000
NI = 50000
E = 800000
DIN = 128
H = 64
DOUT = 32

EP = 819200
ER = EP // 128
NN = 51200
NP = 51200
DUMP = 50000
NTILE = 16
ROWS_T = ER // NTILE
ROWS_C = ER // 2 // NTILE
NBC = 10
NCHC = ROWS_C // NBC
TS = NP // NTILE

BLK = 2048


def _sc_segsum3(w2, nb):
  mesh = plsc.VectorSubcoreMesh(core_axis_name="c", subcore_axis_name="s")
  out1 = jax.ShapeDtypeStruct((2 * NP, w2), jnp.float32)
  nch = ROWS_T // nb
  assert nch % 2 == 0
  buf = lambda: [pltpu.VMEM((nb, 128), jnp.int32),
                 pltpu.VMEM((nb, 128), jnp.int32),
                 pltpu.VMEM((nb, 128, w2), jnp.float32),
                 pltpu.SemaphoreType.DMA,
                 pltpu.SemaphoreType.DMA]

  @functools.partial(
      pl.kernel,
      out_type=(out1, out1, out1),
      mesh=mesh,
      compiler_params=pltpu.CompilerParams(use_tc_tiling_on_sc=False),
      scratch_types=buf() + buf() + [
          pltpu.VMEM_SHARED((NP, w2), jnp.float32),
      ],
  )
  def k(ya, srca, dsta, yb, srcb, dstb, yc, srcc, dstc, zeros_hbm,
        outa, outb, outc,
        srcv0, dstv0, rows0, gsem0, ssem0,
        srcv1, dstv1, rows1, gsem1, ssem1, acc):
    c = lax.axis_index("c")
    s = lax.axis_index("s")
    bufs = ((srcv0, dstv0, rows0, gsem0, ssem0),
            (srcv1, dstv1, rows1, gsem1, ssem1))

    def drain_sc(rows, ssem):
      for j in range(nb):
        pltpu.make_async_copy(rows.at[j], acc.at[pl.ds(0, 128)], ssem).wait()

    for y, src2, dst2, out in ((ya, srca, dsta, outa),
                               (yb, srcb, dstb, outb),
                               (yc, srcc, dstc, outc)):
      pltpu.sync_copy(zeros_hbm.at[pl.ds(0, TS)], acc.at[pl.ds(s * TS, TS)])
      plsc.subcore_barrier()
      src_base = c * ER + s * ROWS_T
      dst_base = s * ROWS_T

      def pair(i2, carry):
        gcps = []
        for b, (srcv, dstv, rows, gsem, ssem) in enumerate(bufs):
          ci = 2 * i2 + b

          @pl.when(i2 > 0)
          def _():
            drain_sc(rows, ssem)

          pltpu.sync_copy(src2.at[pl.ds(src_base + ci * nb, nb)], srcv)
          pltpu.sync_copy(dst2.at[pl.ds(dst_base + ci * nb, nb)], dstv)
          gcps.append([pltpu.async_copy(y.at[srcv.at[j]], rows.at[j], gsem)
                       for j in range(nb)])
        for b, (srcv, dstv, rows, gsem, ssem) in enumerate(bufs):
          for cp in gcps[b]:
            cp.wait()
          for j in range(nb):
            pltpu.async_copy(rows.at[j], acc.at[dstv.at[j]], ssem, add=True)
        return carry

      lax.fori_loop(0, nch // 2, pair, 0)
      for (srcv, dstv, rows, gsem, ssem) in bufs:
        drain_sc(rows, ssem)
      plsc.subcore_barrier()
      pltpu.sync_copy(acc.at[pl.ds(s * TS, TS)],
                      out.at[pl.ds(c * NP + s * TS, TS)])
    return

  return k


def _sc_counts():
  mesh = plsc.VectorSubcoreMesh(core_axis_name="c", subcore_axis_name="s")
  out1 = jax.ShapeDtypeStruct((2 * NP, 16), jnp.float32)

  @functools.partial(
      pl.kernel,
      out_type=(out1, out1, out1),
      mesh=mesh,
      compiler_params=pltpu.CompilerParams(use_tc_tiling_on_sc=False),
      scratch_types=[
          pltpu.VMEM((NBC, 128), jnp.int32),
          pltpu.VMEM((NBC, 128), jnp.int32),
          pltpu.VMEM((128, 16), jnp.float32),
          pltpu.VMEM_SHARED((NP, 16), jnp.float32),
          pltpu.SemaphoreType.DMA,
          pltpu.SemaphoreType.DMA,
      ],
  )
  def k(dsta, dstb, dstc, ones_hbm, zeros_hbm,
        outa, outb, outc, dstv0, dstv1, ones, acc, sem0, sem1):
    c = lax.axis_index("c")
    s = lax.axis_index("s")
    pltpu.sync_copy(ones_hbm, ones)
    bufs = ((dstv0, sem0), (dstv1, sem1))

    def drain(sem):
      for j in range(NBC):
        pltpu.make_async_copy(ones, acc.at[pl.ds(0, 128)], sem).wait()

    for dst2, out in ((dsta, outa), (dstb, outb), (dstc, outc)):
      pltpu.sync_copy(zeros_hbm.at[pl.ds(0, TS)], acc.at[pl.ds(s * TS, TS)])
      plsc.subcore_barrier()
      base = c * (ER // 2) + s * ROWS_C

      def pair(i2, carry):
        for b, (dstv, sem) in enumerate(bufs):
          @pl.when(i2 > 0)
          def _():
            drain(sem)

          pltpu.sync_copy(dst2.at[pl.ds(base + (2 * i2 + b) * NBC, NBC)], dstv)
          for j in range(NBC):
            pltpu.async_copy(ones, acc.at[dstv.at[j]], sem, add=True)
        return carry

      lax.fori_loop(0, NCHC // 2, pair, 0)
      for dstv, sem in bufs:
        drain(sem)
      plsc.subcore_barrier()
      pltpu.sync_copy(acc.at[pl.ds(s * TS, TS)],
                      out.at[pl.ds(c * NP + s * TS, TS)])
    return

  return k


_HI = lax.Precision.HIGHEST


def _full(shape):
  return pl.BlockSpec(shape, lambda i: (0,) * len(shape))


def _dot(a, b):
  return jnp.dot(a, b, preferred_element_type=jnp.float32, precision=_HI)


def _tc_pre_user():
  in_specs = [pl.BlockSpec((BLK, DIN), lambda i: (i, 0)),
              _full((DIN, H)), _full((1, H)),
              _full((H, H)), _full((H, H)), _full((H, H)), _full((1, H))]
  out_shape = (jax.ShapeDtypeStruct((NN, 128), jnp.float32),
               jax.ShapeDtypeStruct((NN, H), jnp.float32))
  out_specs = (pl.BlockSpec((BLK, 128), lambda i: (i, 0)),
               pl.BlockSpec((BLK, H), lambda i: (i, 0)))

  def body(x, pwt, pb, wls, wli, wrt, blc, tuo, duo):
    h = _dot(x[...], pwt[...]) + pb[...]
    tuo[...] = jnp.concatenate([_dot(h, wls[...]), _dot(h, wli[...])], axis=1)
    duo[...] = _dot(h, wrt[...]) + blc[...]

  return pl.pallas_call(body, grid=(NN // BLK,), in_specs=in_specs,
                        out_specs=out_specs, out_shape=out_shape)


def _tc_pre_item():
  in_specs = [pl.BlockSpec((BLK, DIN), lambda i: (i, 0)),
              _full((DIN, H)), _full((1, H)),
              _full((H, H)), _full((H, H)), _full((1, H))]
  out_shape = jax.ShapeDtypeStruct((NN, 128), jnp.float32)
  out_specs = pl.BlockSpec((BLK, 128), lambda i: (i, 0))

  def body(x, pwt, pb, wlr, wri, bli, tio):
    h = _dot(x[...], pwt[...]) + pb[...]
    tio[...] = jnp.concatenate([_dot(h, wlr[...]),
                                _dot(h, wri[...]) + bli[...]], axis=1)

  return pl.pallas_call(body, grid=(NN // BLK,), in_specs=in_specs,
                        out_specs=out_specs, out_shape=out_shape)


def _seg_spec(w2):
  return pl.BlockSpec((2, BLK, w2), lambda i: (0, i, 0))


def _agg(p_ref, c_ref):
  inv = 1.0 / jnp.maximum(c_ref[0, :, 0:1] + c_ref[1, :, 0:1], 1.0)
  return jnp.concatenate([p_ref[0], p_ref[1]], axis=1) * inv


def _tc_mid_user():
  in_specs = [_seg_spec(32), _seg_spec(16), _seg_spec(32), _seg_spec(16),
              pl.BlockSpec((BLK, H), lambda i: (i, 0)),
              _full((H, DOUT)), _full((H, DOUT)),
              _full((H, DOUT)), _full((1, DOUT))]
  out_shape = jax.ShapeDtypeStruct((NN, 128), jnp.float32)
  out_specs = pl.BlockSpec((BLK, 128), lambda i: (i, 0))

  def body(sS, cS, sR, cR, du, wls, wli, wrc, blc, tuo):
    h1 = jnp.maximum((_agg(sS, cS) + _agg(sR, cR) + du[...]) * 0.5, 0.0)
    tuo[...] = jnp.concatenate(
        [_dot(h1, wls[...]), _dot(h1, wli[...]),
         _dot(h1, wrc[...]) + blc[...],
         jnp.zeros((BLK, DOUT), jnp.float32)], axis=1)

  return pl.pallas_call(body, grid=(NN // BLK,), in_specs=in_specs,
                        out_specs=out_specs, out_shape=out_shape)


def _tc_mid_item():
  in_specs = [_seg_spec(32), _seg_spec(16),
              pl.BlockSpec((BLK, 128), lambda i: (i, 0)),
              _full((H, DOUT)), _full((H, DOUT)), _full((1, DOUT))]
  out_shape = jax.ShapeDtypeStruct((NN, 128), jnp.float32)
  out_specs = pl.BlockSpec((BLK, 128), lambda i: (i, 0))

  def body(sI, cI, ti, wlr, wri, bli, tio):
    h1 = jnp.maximum(_agg(sI, cI) + ti[:, H:], 0.0)
    tio[...] = jnp.concatenate(
        [_dot(h1, wlr[...]), _dot(h1, wri[...]) + bli[...],
         jnp.zeros((BLK, 2 * DOUT), jnp.float32)], axis=1)

  return pl.pallas_call(body, grid=(NN // BLK,), in_specs=in_specs,
                        out_specs=out_specs, out_shape=out_shape)


def _tc_post_user():
  in_specs = [_seg_spec(16), _seg_spec(16), _seg_spec(16), _seg_spec(16),
              pl.BlockSpec((BLK, 128), lambda i: (i, 0))]
  out_shape = jax.ShapeDtypeStruct((NN, DOUT), jnp.float32)
  out_specs = pl.BlockSpec((BLK, DOUT), lambda i: (i, 0))

  def body(sS, cS, sR, cR, tu2, o):
    du2 = tu2[:, 64:96]
    o[...] = jnp.maximum((_agg(sS, cS) + _agg(sR, cR) + du2) * 0.5, 0.0)

  return pl.pallas_call(body, grid=(NN // BLK,), in_specs=in_specs,
                        out_specs=out_specs, out_shape=out_shape)


def _tc_post_item():
  in_specs = [_seg_spec(16), _seg_spec(16),
              pl.BlockSpec((BLK, 128), lambda i: (i, 0))]
  out_shape = jax.ShapeDtypeStruct((NN, DOUT), jnp.float32)
  out_specs = pl.BlockSpec((BLK, DOUT), lambda i: (i, 0))

  def body(sI, cI, ti2, o):
    di2 = ti2[:, 32:64]
    o[...] = jnp.maximum(_agg(sI, cI) + di2, 0.0)

  return pl.pallas_call(body, grid=(NN // BLK,), in_specs=in_specs,
                        out_specs=out_specs, out_shape=out_shape)


def _prep_edges(ei, q1, q2):
  src = ei[0].astype(jnp.int32)
  dst = ei[1].astype(jnp.int32)
  pad = EP - E
  src = jnp.concatenate([src, jnp.zeros((pad,), jnp.int32)])
  dst = jnp.concatenate([dst, jnp.full((pad,), DUMP, jnp.int32)])
  s4 = src * 4 + q1
  s8 = src * 8 + q2
  src2_1 = jnp.concatenate([s4, s4 + 1]).reshape(2 * ER, 128)
  src2_2 = jnp.concatenate([s8, s8 + 1]).reshape(2 * ER, 128)
  return src2_1, src2_2, dst.reshape(ER, 128)


def kernel(x_user, x_item, edge_index_social, edge_index_interacts,
           edge_index_rev_interacts, up_W, up_b, ip_W, ip_b,
           c1s_Wl, c1s_bl, c1s_Wr, c1i_Wl, c1i_bl, c1i_Wr,
           c1r_Wl, c1r_bl, c1r_Wr,
           c2s_Wl, c2s_bl, c2s_Wr, c2i_Wl, c2i_bl, c2i_Wr,
           c2r_Wl, c2r_bl, c2r_Wr):
  sa_s, sb_s, dst2_s = _prep_edges(edge_index_social, 0, 0)
  sa_i, sb_i, dst2_i = _prep_edges(edge_index_interacts, 2, 2)
  sa_r, sb_r, dst2_r = _prep_edges(edge_index_rev_interacts, 0, 0)

  zeros32 = jnp.zeros((TS, 32), jnp.float32)
  zeros16 = jnp.zeros((TS, 16), jnp.float32)
  ones128 = jnp.ones((128, 16), jnp.float32)

  xpad = jnp.zeros((NN - NU, DIN), jnp.float32)
  xu = jnp.concatenate([x_user, xpad])
  xi = jnp.concatenate([x_item, xpad])

  tu, du1 = _tc_pre_user()(
      xu, up_W.T, up_b.reshape(1, H),
      c1s_Wl.T, c1i_Wl.T,
      (c1s_Wr + c1r_Wr).T, (c1s_bl + c1r_bl).reshape(1, H))
  ti = _tc_pre_item()(
      xi, ip_W.T, ip_b.reshape(1, H),
      c1r_Wl.T, c1i_Wr.T, c1i_bl.reshape(1, H))

  cnt_s, cnt_i, cnt_r = _sc_counts()(dst2_s, dst2_i, dst2_r, ones128, zeros16)
  tu4 = tu.reshape(4 * NN, 32)
  ti4 = ti.reshape(4 * NN, 32)
  seg_s, seg_i, seg_r = _sc_segsum3(32, 2)(
      tu4, sa_s, dst2_s,
      tu4, sa_i, dst2_i,
      ti4, sa_r, dst2_r,
      zeros32)

  r2 = lambda a, w: a.reshape(2, NP, w)
  tu2 = _tc_mid_user()(
      r2(seg_s, 32), r2(cnt_s, 16), r2(seg_r, 32), r2(cnt_r, 16),
      du1,
      c2s_Wl.T, c2i_Wl.T,
      (c2s_Wr + c2r_Wr).T, (c2s_bl + c2r_bl).reshape(1, DOUT))
  ti2 = _tc_mid_item()(
      r2(seg_i, 32), r2(cnt_i, 16), ti,
      c2r_Wl.T, c2i_Wr.T, c2i_bl.reshape(1, DOUT))

  tu8 = tu2.reshape(8 * NN, 16)
  ti8 = ti2.reshape(8 * NN, 16)
  s2_s, s2_i, s2_r = _sc_segsum3(16, 8)(
      tu8, sb_s, dst2_s,
      tu8, sb_i, dst2_i,
      ti8, sb_r, dst2_r,
      zeros16)

  h2u = _tc_post_user()(
      r2(s2_s, 16), r2(cnt_s, 16), r2(s2_r, 16), r2(cnt_r, 16), tu2)
  h2i = _tc_post_item()(
      r2(s2_i, 16), r2(cnt_i, 16), ti2)
  return (h2u[:NU], h2i[:NI])
